# baseline (speedup 1.0000x reference)
import jax
import jax.numpy as jnp
from jax import lax
from jax.experimental import pallas as pl
from jax.experimental.pallas import tpu as pltpu
from jax.experimental.pallas import tpu_sc as plsc

N, C, H, W = 8, 96, 64, 64
K2 = 9
P = H * W
PW = W + 2

NUM_WORKERS = 32
NB = 4
POS_PER_W = NB * P // NUM_WORKERS
CPOS = 16
CROWS = CPOS * K2
GDMA = 72
NCHUNKS = POS_PER_W // CPOS
NBUF = 4
LANES = 16


def _sc_body(table_hbm, cnt_hbm, out_hbm, cnt_v, idx_v, rows_v, *sems):
    gsems = sems[:NBUF]
    ssems = sems[NBUF:]
    num_cores = 2
    wid = lax.axis_index("s") * num_cores + lax.axis_index("c")
    workers_per_batch = NUM_WORKERS // NB
    n = wid // workers_per_batch
    nbase = n * P
    p0base = (wid % workers_per_batch) * POS_PER_W
    cntbase = p0base * K2

    pltpu.sync_copy(cnt_hbm.at[pl.ds(cntbase, POS_PER_W * K2)], cnt_v)

    vpw = jnp.full((LANES,), PW, jnp.int32)
    vone = jnp.full((LANES,), 1, jnp.int32)
    vzero = jnp.full((LANES,), 0, jnp.int32)
    vhmax = jnp.full((LANES,), H - 1, jnp.int32)
    vwmax = jnp.full((LANES,), W - 1, jnp.int32)
    vw = jnp.full((LANES,), W, jnp.int32)
    vnb = lax.broadcast_in_dim(nbase, (LANES,), ())

    def decode(c, b):
        for i in range(CROWS // LANES):
            v = cnt_v[pl.ds(c * CROWS + i * LANES, LANES)]
            ph = lax.div(v, vpw)
            pw = lax.sub(v, lax.mul(ph, vpw))
            hh = lax.max(lax.min(lax.sub(ph, vone), vhmax), vzero)
            ww = lax.max(lax.min(lax.sub(pw, vone), vwmax), vzero)
            idx_v[b, pl.ds(i * LANES, LANES)] = lax.add(
                lax.add(lax.mul(hh, vw), ww), vnb)


    def gather_start(b):
        for j in range(CROWS // GDMA):
            pltpu.async_copy(
                table_hbm.at[idx_v.at[b, pl.ds(j * GDMA, GDMA)]],
                rows_v.at[b, pl.ds(j * GDMA, GDMA), :], gsems[b])

    def gather_wait(b):
        for j in range(CROWS // GDMA):
            pltpu.make_async_copy(
                table_hbm.at[idx_v.at[b, pl.ds(j * GDMA, GDMA)]],
                rows_v.at[b, pl.ds(j * GDMA, GDMA), :], gsems[b]).wait()

    def scatter_start(b, g):
        for k in range(K2):
            pltpu.async_copy(
                rows_v.at[b, pl.ds(k * CPOS, CPOS), :],
                out_hbm.at[n, pl.ds(p0base + g * CPOS, CPOS),
                           pl.ds(k * C, C)], ssems[b])

    def scatter_wait(b):
        for k in range(K2):
            pltpu.make_async_copy(
                rows_v.at[b, pl.ds(k * CPOS, CPOS), :],
                out_hbm.at[n, pl.ds(p0base, CPOS), pl.ds(k * C, C)],
                ssems[b]).wait()

    for b in range(NBUF):
        decode(b, b)
        gather_start(b)

    def outer_body(go, carry):
        for b in range(NBUF):
            g = go * NBUF + b
            gather_wait(b)
            scatter_start(b, g)
            decode(g + NBUF, b)
            scatter_wait(b)
            gather_start(b)
        return carry

    lax.fori_loop(0, (NCHUNKS - NBUF) // NBUF, outer_body, 0)

    for b in range(NBUF):
        g = NCHUNKS - NBUF + b
        gather_wait(b)
        scatter_start(b, g)
    for b in range(NBUF):
        scatter_wait(b)


@jax.jit
def _spatialflatten_sc(table, cnt):
    mesh = plsc.VectorSubcoreMesh(core_axis_name="c", subcore_axis_name="s")
    fn = pl.kernel(
        _sc_body,
        out_type=jax.ShapeDtypeStruct((NB, P, K2 * C), jnp.float32),
        mesh=mesh,
        scratch_types=[
            pltpu.VMEM((POS_PER_W * K2,), jnp.int32),
            pltpu.VMEM((NBUF, CROWS), jnp.int32),
            pltpu.VMEM((NBUF, CROWS, C), jnp.float32),
        ] + [pltpu.SemaphoreType.DMA] * (2 * NBUF),
        compiler_params=pltpu.CompilerParams(use_tc_tiling_on_sc=False),
    )
    return fn(table, cnt)


def kernel(fm, counts):
    table = jnp.transpose(fm, (0, 2, 3, 1)).reshape(N * P, C)
    cnt = (counts.astype(jnp.int32).reshape(P // CPOS, CPOS, K2)
           .swapaxes(1, 2).reshape(P * K2))
    halves = [_spatialflatten_sc(table[s * NB * P:(s + 1) * NB * P], cnt)
              for s in range(N // NB)]
    return jnp.concatenate(halves, axis=0)


# speedup vs baseline: 1.1710x; 1.1710x over previous
import jax
import jax.numpy as jnp
from jax import lax
from jax.experimental import pallas as pl
from jax.experimental.pallas import tpu as pltpu
from jax.experimental.pallas import tpu_sc as plsc

N, C, H, W = 8, 96, 64, 64
K2 = 9
P = H * W
PW = W + 2

NUM_WORKERS = 32
POS_PER_W = N * P // NUM_WORKERS
CPOS = 16
CROWS = CPOS * K2
GDMA = 72
NCHUNKS = POS_PER_W // CPOS
NBUF = 4
LANES = 16


def _sc_body(table_hbm, cnt_hbm, out_hbm, cnt_v, idx_v, rows_v, *sems):
    gsems = sems[:NBUF]
    ssems = sems[NBUF:]
    num_cores = 2
    wid = lax.axis_index("s") * num_cores + lax.axis_index("c")
    workers_per_batch = NUM_WORKERS // N
    n = wid // workers_per_batch
    nbase = n * P
    p0base = (wid % workers_per_batch) * POS_PER_W
    cntbase = p0base * K2

    pltpu.sync_copy(cnt_hbm.at[pl.ds(cntbase, POS_PER_W * K2)], cnt_v)

    vpw = jnp.full((LANES,), PW, jnp.int32)
    vone = jnp.full((LANES,), 1, jnp.int32)
    vzero = jnp.full((LANES,), 0, jnp.int32)
    vhmax = jnp.full((LANES,), H - 1, jnp.int32)
    vwmax = jnp.full((LANES,), W - 1, jnp.int32)
    vw = jnp.full((LANES,), W, jnp.int32)
    vnb = lax.broadcast_in_dim(nbase, (LANES,), ())

    def decode(c, b):
        for i in range(CROWS // LANES):
            v = cnt_v[pl.ds(c * CROWS + i * LANES, LANES)]
            ph = lax.div(v, vpw)
            pw = lax.sub(v, lax.mul(ph, vpw))
            hh = lax.max(lax.min(lax.sub(ph, vone), vhmax), vzero)
            ww = lax.max(lax.min(lax.sub(pw, vone), vwmax), vzero)
            idx_v[b, pl.ds(i * LANES, LANES)] = lax.add(
                lax.add(lax.mul(hh, vw), ww), vnb)


    def gather_start(b):
        for j in range(CROWS // GDMA):
            pltpu.async_copy(
                table_hbm.at[idx_v.at[b, pl.ds(j * GDMA, GDMA)]],
                rows_v.at[b, pl.ds(j * GDMA, GDMA), :], gsems[b])

    def gather_wait(b):
        for j in range(CROWS // GDMA):
            pltpu.make_async_copy(
                table_hbm.at[idx_v.at[b, pl.ds(j * GDMA, GDMA)]],
                rows_v.at[b, pl.ds(j * GDMA, GDMA), :], gsems[b]).wait()

    def scatter_start(b, g):
        for k in range(K2):
            pltpu.async_copy(
                rows_v.at[b, pl.ds(k * CPOS, CPOS), :],
                out_hbm.at[n, pl.ds(p0base + g * CPOS, CPOS),
                           pl.ds(k * C, C)], ssems[b])

    def scatter_wait(b):
        for k in range(K2):
            pltpu.make_async_copy(
                rows_v.at[b, pl.ds(k * CPOS, CPOS), :],
                out_hbm.at[n, pl.ds(p0base, CPOS), pl.ds(k * C, C)],
                ssems[b]).wait()

    for b in range(NBUF):
        decode(b, b)
        gather_start(b)

    def outer_body(go, carry):
        for b in range(NBUF):
            g = go * NBUF + b
            gather_wait(b)
            scatter_start(b, g)
            decode(g + NBUF, b)
            scatter_wait(b)
            gather_start(b)
        return carry

    lax.fori_loop(0, (NCHUNKS - NBUF) // NBUF, outer_body, 0)

    for b in range(NBUF):
        g = NCHUNKS - NBUF + b
        gather_wait(b)
        scatter_start(b, g)
    for b in range(NBUF):
        scatter_wait(b)


@jax.jit
def _spatialflatten_sc(table, cnt):
    mesh = plsc.VectorSubcoreMesh(core_axis_name="c", subcore_axis_name="s")
    fn = pl.kernel(
        _sc_body,
        out_type=jax.ShapeDtypeStruct((N, P, K2 * C), jnp.float32),
        mesh=mesh,
        scratch_types=[
            pltpu.VMEM((POS_PER_W * K2,), jnp.int32),
            pltpu.VMEM((NBUF, CROWS), jnp.int32),
            pltpu.VMEM((NBUF, CROWS, C), jnp.float32),
        ] + [pltpu.SemaphoreType.DMA] * (2 * NBUF),
        compiler_params=pltpu.CompilerParams(use_tc_tiling_on_sc=False),
    )
    return fn(table, cnt)


def kernel(fm, counts):
    table = jnp.transpose(fm, (0, 2, 3, 1)).reshape(N * P, C)
    cnt = (counts.astype(jnp.int32).reshape(P // CPOS, CPOS, K2)
           .swapaxes(1, 2).reshape(P * K2))
    return _spatialflatten_sc(table, cnt)
